# Initial kernel scaffold; baseline (speedup 1.0000x reference)
import jax
import jax.numpy as jnp
from jax.experimental import pallas as pl


def kernel(x, pos, edge_index, Wh1, bh1, Wh2, bh2, Wg1, bg1, Wg2, bg2):
    raise NotImplementedError("write your pallas kernel here")


# speedup vs baseline: 15.4682x; 15.4682x over previous
import functools

import jax
import jax.numpy as jnp
from jax import lax
from jax.experimental import pallas as pl
from jax.experimental.pallas import tpu as pltpu
from jax.experimental.pallas import tpu_sc as plsc

D = 128
WP = 16
BN = 512
CH = 80
NSUB = 16


def _sc_segment_sums(T, Tp, src3, dst3, nc, niter, np_rows):
    rps = np_rows // NSUB
    mesh = plsc.VectorSubcoreMesh(core_axis_name="c", subcore_axis_name="s")

    @functools.partial(
        pl.kernel,
        out_type=(
            jax.ShapeDtypeStruct((nc, np_rows, D), jnp.float32),
            jax.ShapeDtypeStruct((nc, np_rows, WP), jnp.float32),
        ),
        mesh=mesh,
        compiler_params=pltpu.CompilerParams(use_tc_tiling_on_sc=False),
        scratch_types=[
            pltpu.VMEM((niter, CH), jnp.int32),
            pltpu.VMEM((niter, CH), jnp.int32),
            pltpu.VMEM((CH, D), jnp.float32),
            pltpu.VMEM((CH, WP), jnp.float32),
            pltpu.VMEM_SHARED((np_rows, D), jnp.float32),
            pltpu.VMEM_SHARED((np_rows, WP), jnp.float32),
            pltpu.SemaphoreType.DMA,
            pltpu.SemaphoreType.DMA,
        ],
    )
    def sc_kernel(t_hbm, tp_hbm, src_hbm, dst_hbm, outx_hbm, outp_hbm,
                  src_v, dst_v, rowsx_v, rowsp_v, accx, accp, sem1, sem2):
        c = lax.axis_index("c")
        s = lax.axis_index("s")
        wid = s * nc + c

        zero16 = jnp.zeros((16,), jnp.float32)

        def zbody(i, carry):
            r = i // (D // 16)
            col = (i % (D // 16)) * 16
            rowsx_v[r, pl.ds(col, 16)] = zero16
            return carry

        lax.fori_loop(0, CH * (D // 16), zbody, 0)

        def zbody2(i, carry):
            rowsp_v[i, pl.ds(0, 16)] = zero16
            return carry

        lax.fori_loop(0, CH, zbody2, 0)

        for k in range(rps // CH):
            pltpu.sync_copy(rowsx_v, accx.at[pl.ds(s * rps + k * CH, CH)])
            pltpu.sync_copy(rowsp_v, accp.at[pl.ds(s * rps + k * CH, CH)])
        plsc.subcore_barrier()

        pltpu.sync_copy(src_hbm.at[wid], src_v)
        pltpu.sync_copy(dst_hbm.at[wid], dst_v)

        def body(j, carry):
            cpx = pltpu.async_copy(t_hbm.at[src_v.at[j]], rowsx_v, sem1)
            cpp = pltpu.async_copy(tp_hbm.at[src_v.at[j]], rowsp_v, sem2)
            cpx.wait()
            cpp.wait()
            pltpu.sync_copy(rowsx_v, accx.at[dst_v.at[j]], add=True)
            pltpu.sync_copy(rowsp_v, accp.at[dst_v.at[j]], add=True)
            return carry

        lax.fori_loop(0, niter, body, 0)
        plsc.subcore_barrier()

        pltpu.sync_copy(accx.at[pl.ds(s * rps, rps)],
                        outx_hbm.at[c, pl.ds(s * rps, rps)])
        pltpu.sync_copy(accp.at[pl.ds(s * rps, rps)],
                        outp_hbm.at[c, pl.ds(s * rps, rps)])

    return sc_kernel(T, Tp, src3, dst3)


def _combine(xp, pos16, Px, Pp, Wh1, bh1r, Wh2p, bh2r, Wg1p, Wg1x, bg1r,
             Wg2, bg2r, np_rows):
    nb = np_rows // BN

    def body(x_ref, pos_ref, px_ref, pp_ref, wh1, bh1, wh2, bh2, wg1p, wg1x,
             bg1, wg2, bg2, o_ref):
        x = x_ref[...]
        h1 = jnp.maximum(
            jnp.dot(x, wh1[...], preferred_element_type=jnp.float32)
            + bh1[...], 0.0)
        delta = jnp.tanh(
            jnp.dot(h1, wh2[...], preferred_element_type=jnp.float32)
            + bh2[...])
        Sx = px_ref[0] + px_ref[1]
        Sp = pp_ref[0] + pp_ref[1]
        cnt = Sp[:, 3:4]
        inv = 1.0 / jnp.maximum(cnt, 1.0)
        aggr_x = Sx * inv
        aggr_p = (Sp - cnt * (pos_ref[...] - delta[:, :16])) * inv
        g = jnp.maximum(
            jnp.dot(aggr_p, wg1p[...], preferred_element_type=jnp.float32)
            + jnp.dot(aggr_x, wg1x[...], preferred_element_type=jnp.float32)
            + bg1[...], 0.0)
        out = jnp.maximum(
            jnp.dot(g, wg2[...], preferred_element_type=jnp.float32)
            + bg2[...], 0.0)
        o_ref[...] = x + out

    return pl.pallas_call(
        body,
        grid=(nb,),
        in_specs=[
            pl.BlockSpec((BN, D), lambda i: (i, 0)),
            pl.BlockSpec((BN, 16), lambda i: (i, 0)),
            pl.BlockSpec((2, BN, D), lambda i: (0, i, 0)),
            pl.BlockSpec((2, BN, WP), lambda i: (0, i, 0)),
            pl.BlockSpec((D, D), lambda i: (0, 0)),
            pl.BlockSpec((1, D), lambda i: (0, 0)),
            pl.BlockSpec((D, D), lambda i: (0, 0)),
            pl.BlockSpec((1, D), lambda i: (0, 0)),
            pl.BlockSpec((16, D), lambda i: (0, 0)),
            pl.BlockSpec((D, D), lambda i: (0, 0)),
            pl.BlockSpec((1, D), lambda i: (0, 0)),
            pl.BlockSpec((D, D), lambda i: (0, 0)),
            pl.BlockSpec((1, D), lambda i: (0, 0)),
        ],
        out_specs=pl.BlockSpec((BN, D), lambda i: (i, 0)),
        out_shape=jax.ShapeDtypeStruct((np_rows, D), jnp.float32),
    )(xp, pos16, Px, Pp, Wh1, bh1r, Wh2p, bh2r, Wg1p, Wg1x, bg1r, Wg2, bg2r)


def kernel(x, pos, edge_index, Wh1, bh1, Wh2, bh2, Wg1, bg1, Wg2, bg2):
    N = x.shape[0]
    E = edge_index.shape[1]
    nc = plsc.get_sparse_core_info().num_cores
    nw = nc * NSUB

    np_rows = (N // BN + 1) * BN

    xp = jnp.pad(x, ((0, np_rows - N), (0, 0)))
    Tp = jnp.concatenate([pos, jnp.ones((N, 1), jnp.float32)], axis=1)
    Tp = jnp.pad(Tp, ((0, np_rows - N), (0, WP - 4)))
    pos16 = jnp.pad(pos, ((0, np_rows - N), (0, 13)))

    ep = -(-E // (nw * CH)) * (nw * CH)
    src = edge_index[0]
    dst = edge_index[1]
    if ep != E:
        pad = jnp.full((ep - E,), N, jnp.int32)
        src = jnp.concatenate([src, pad])
        dst = jnp.concatenate([dst, pad])
    niter = ep // (nw * CH)
    src3 = src.reshape(nw, niter, CH)
    dst3 = dst.reshape(nw, niter, CH)

    Px, Pp = _sc_segment_sums(xp, Tp, src3, dst3, nc, niter, np_rows)

    bh1r = bh1.reshape(1, D)
    bh2r = jnp.pad(bh2, (0, D - 3)).reshape(1, D)
    bg1r = bg1.reshape(1, D)
    bg2r = bg2.reshape(1, D)
    Wh2p = jnp.pad(Wh2, ((0, 0), (0, D - 3)))
    Wg1p = jnp.pad(Wg1[:3], ((0, 13), (0, 0)))
    Wg1x = Wg1[3:]

    y = _combine(xp, pos16, Px, Pp, Wh1, bh1r, Wh2p, bh2r, Wg1p, Wg1x, bg1r,
                 Wg2, bg2r, np_rows)
    return y[:N]
